# baseline (speedup 1.0000x reference)
import functools

import jax
import jax.numpy as jnp
from jax import lax
from jax.experimental import pallas as pl
from jax.experimental.pallas import tpu as pltpu
from jax.experimental.pallas import tpu_sc as plsc

N = 10000
D = 128
EPS = 1e-5

NC = 2
NS = 16
L = 16
NW = NC * NS
K = 128
CW = 8

NPAD = 10240
ROWS_PER_TILE = NPAD // NS
TRASH = N + 64


def _fill(ref, value):
  @pl.loop(0, K)
  def _(i):
    for j in range(D // L):
      ref[i, pl.ds(j * L, L)] = jnp.full((L,), value, jnp.float32)


def _make_sc_agg(n_chunks):
  mesh = plsc.VectorSubcoreMesh(core_axis_name="c", subcore_axis_name="s")

  def body(h_hbm, src_hbm, dst_hbm, agg_out,
           sidx_v, dst_v, rows_v, agg_sh, sem_g, sem_i):
    cid = lax.axis_index("c")
    sid = lax.axis_index("s")
    base = sid * ROWS_PER_TILE
    cbase = (cid * NS + sid) * n_chunks

    _fill(rows_v.at[0], 0.0)
    for b in range(ROWS_PER_TILE // K):
      pltpu.sync_copy(rows_v.at[0], agg_sh.at[pl.ds(base + b * K, K)])
    plsc.subcore_barrier()

    pltpu.sync_copy(dst_hbm.at[0, pl.ds(cbase, n_chunks)], dst_v)
    pltpu.sync_copy(src_hbm.at[0, cbase], sidx_v.at[0])
    pltpu.async_copy(h_hbm.at[sidx_v.at[0]], rows_v.at[0], sem_g)
    pltpu.async_copy(src_hbm.at[0, cbase + 1], sidx_v.at[1], sem_i)

    @pl.loop(0, n_chunks)
    def _(j):
      cur = lax.rem(j, 2)
      nxt = lax.rem(j + 1, 2)
      s_cur = lax.rem(j, 4)
      s_nxt = lax.rem(j + 1, 4)
      s_pre = lax.rem(j + 2, 4)
      pltpu.make_async_copy(h_hbm.at[sidx_v.at[s_cur]], rows_v.at[cur],
                            sem_g).wait()

      @pl.when(j + 1 < n_chunks)
      def _():
        pltpu.make_async_copy(src_hbm.at[0, cbase], sidx_v.at[s_nxt],
                              sem_i).wait()
        pltpu.async_copy(h_hbm.at[sidx_v.at[s_nxt]], rows_v.at[nxt], sem_g)

        @pl.when(j + 2 < n_chunks)
        def _():
          pltpu.async_copy(src_hbm.at[0, cbase + j + 2], sidx_v.at[s_pre],
                           sem_i)

      pltpu.sync_copy(rows_v.at[cur], agg_sh.at[dst_v.at[j]], add=True)

    plsc.subcore_barrier()
    pltpu.sync_copy(agg_sh.at[pl.ds(base, ROWS_PER_TILE)],
                    agg_out.at[cid, pl.ds(base, ROWS_PER_TILE)])

  return pl.kernel(
      body,
      out_type=jax.ShapeDtypeStruct((NC, NPAD, D), jnp.float32),
      mesh=mesh,
      scratch_types=[
          pltpu.VMEM((4, K), jnp.int32),
          pltpu.VMEM((n_chunks, K), jnp.int32),
          pltpu.VMEM((2, K, D), jnp.float32),
          pltpu.VMEM_SHARED((NPAD, D), jnp.float32),
          pltpu.SemaphoreType.DMA,
          pltpu.SemaphoreType.DMA,
      ],
  )


def _make_sc_count(n_chunks):
  mesh = plsc.VectorSubcoreMesh(core_axis_name="c", subcore_axis_name="s")

  def body(dst_hbm, cnt_out, dst_v, buf_v, cnt_sh):
    cid = lax.axis_index("c")
    sid = lax.axis_index("s")
    w = cid * NS + sid
    base = sid * ROWS_PER_TILE

    _fill(buf_v, 0.0)
    for b in range(ROWS_PER_TILE // K):
      pltpu.sync_copy(buf_v, cnt_sh.at[pl.ds(base + b * K, K)])
    plsc.subcore_barrier()

    _fill(buf_v, 1.0)
    pltpu.sync_copy(dst_hbm.at[0, pl.ds(w * n_chunks, n_chunks)], dst_v)

    @pl.loop(0, n_chunks)
    def _(j):
      pltpu.sync_copy(buf_v, cnt_sh.at[dst_v.at[j]], add=True)

    plsc.subcore_barrier()
    pltpu.sync_copy(cnt_sh.at[pl.ds(base, ROWS_PER_TILE)],
                    cnt_out.at[cid, pl.ds(base, ROWS_PER_TILE)])

  return pl.kernel(
      body,
      out_type=jax.ShapeDtypeStruct((NC, NPAD, D), jnp.float32),
      mesh=mesh,
      scratch_types=[
          pltpu.VMEM((n_chunks, K), jnp.int32),
          pltpu.VMEM((K, D), jnp.float32),
          pltpu.VMEM_SHARED((NPAD, D), jnp.float32),
      ],
  )


_B = 1024


def _tc_layer_body(a0, a1, c0, c1, h, wl, bl, wr, g, be, out):
  cnt = c0[...][:, 0:1] + c1[...][:, 0:1]
  inv = 1.0 / jnp.maximum(cnt, 1.0)
  mean = (a0[...] + a1[...]) * inv
  z = (jnp.dot(mean, wl[...], preferred_element_type=jnp.float32) + bl[...]
       + jnp.dot(h[...], wr[...], preferred_element_type=jnp.float32))
  scale = g[...] * lax.rsqrt(jnp.float32(1.0 + EPS))
  z = z * scale + be[...]
  out[...] = jnp.maximum(z, 0.0) + h[...]


def _tc_final_body(a0, a1, c0, c1, h, wl, bl, wr, out):
  cnt = c0[...][:, 0:1] + c1[...][:, 0:1]
  inv = 1.0 / jnp.maximum(cnt, 1.0)
  mean = (a0[...] + a1[...]) * inv
  out[...] = (jnp.dot(mean, wl[...], preferred_element_type=jnp.float32)
              + bl[...]
              + jnp.dot(h[...], wr[...], preferred_element_type=jnp.float32))


def _row_spec():
  return pl.BlockSpec((_B, D), lambda i: (i, 0))


def _cnt_spec():
  return pl.BlockSpec((_B, CW), lambda i: (i, 0))


def _full_spec():
  return pl.BlockSpec((D, D), lambda i: (0, 0))


def _vec_spec():
  return pl.BlockSpec((1, D), lambda i: (0, 0))


def _make_tc_layer():
  return pl.pallas_call(
      _tc_layer_body,
      grid=(NPAD // _B,),
      in_specs=[_row_spec(), _row_spec(), _cnt_spec(), _cnt_spec(), _row_spec(),
                _full_spec(), _vec_spec(), _full_spec(), _vec_spec(), _vec_spec()],
      out_specs=_row_spec(),
      out_shape=jax.ShapeDtypeStruct((NPAD, D), jnp.float32),
  )


def _make_tc_final():
  return pl.pallas_call(
      _tc_final_body,
      grid=(NPAD // _B,),
      in_specs=[_row_spec(), _row_spec(), _cnt_spec(), _cnt_spec(), _row_spec(),
                _full_spec(), _vec_spec(), _full_spec()],
      out_specs=_row_spec(),
      out_shape=jax.ShapeDtypeStruct((NPAD, D), jnp.float32),
  )


def kernel(x, edge_index, Wl0, bl0, Wr0, Wl1, bl1, Wr1, Wl2, bl2, Wr2,
           g0, be0, g1, be1):
  e = edge_index.shape[1]
  n_chunks = 80
  e_pad = NW * n_chunks * K
  n_max = n_chunks
  src = jnp.concatenate(
      [edge_index[0], jnp.zeros((e_pad - e,), jnp.int32)]).reshape(-1, K)
  trash = TRASH + jnp.arange(e_pad - e, dtype=jnp.int32) % (NPAD - TRASH)
  dst = jnp.concatenate([edge_index[1], trash]).reshape(-1, K)
  src = jnp.concatenate([src, jnp.zeros((n_max, K), jnp.int32)])[None]
  dst = jnp.concatenate([dst, jnp.full((n_max, K), TRASH, jnp.int32)])[None]
  del n_max
  xp = jnp.concatenate([x, jnp.zeros((NPAD - N, D), x.dtype)], axis=0)

  sc_agg = _make_sc_agg(n_chunks)
  sc_count = _make_sc_count(n_chunks)
  tc_layer = _make_tc_layer()
  tc_final = _make_tc_final()

  bl0r, g0r, be0r = bl0.reshape(1, D), g0.reshape(1, D), be0.reshape(1, D)
  bl1r, g1r, be1r = bl1.reshape(1, D), g1.reshape(1, D), be1.reshape(1, D)
  wl2p = jnp.pad(Wl2, ((0, 0), (0, D - Wl2.shape[1])))
  wr2p = jnp.pad(Wr2, ((0, 0), (0, D - Wr2.shape[1])))
  bl2p = jnp.pad(bl2, (0, D - bl2.shape[0])).reshape(1, D)

  cntw = sc_count(dst)
  cnt = cntw[:, :, :CW]
  agg = sc_agg(xp, src, dst)
  h1 = tc_layer(agg[0], agg[1], cnt[0], cnt[1], xp, Wl0, bl0r, Wr0, g0r, be0r)
  agg1 = sc_agg(h1, src, dst)
  h2 = tc_layer(agg1[0], agg1[1], cnt[0], cnt[1], h1, Wl1, bl1r, Wr1, g1r, be1r)
  agg2 = sc_agg(h2, src, dst)
  outp = tc_final(agg2[0], agg2[1], cnt[0], cnt[1], h2, wl2p, bl2p, wr2p)
  return outp[:N, :2]


# speedup vs baseline: 1.5519x; 1.5519x over previous
import functools

import jax
import jax.numpy as jnp
from jax import lax
from jax.experimental import pallas as pl
from jax.experimental.pallas import tpu as pltpu
from jax.experimental.pallas import tpu_sc as plsc

N = 10000
D = 128
EPS = 1e-5

NC = 2
NS = 16
L = 16
NW = NC * NS
K = 128
CW = 8

NPAD = 10240
ROWS_PER_TILE = NPAD // NS
TRASH = N + 64


def _fill(ref, value):
  @pl.loop(0, K)
  def _(i):
    for j in range(D // L):
      ref[i, pl.ds(j * L, L)] = jnp.full((L,), value, jnp.float32)


def _make_sc_agg(n_chunks):
  mesh = plsc.VectorSubcoreMesh(core_axis_name="c", subcore_axis_name="s")

  def body(h_hbm, src_hbm, dst_hbm, agg_out,
           sidx_v, dst_v, rows_v, agg_sh, sem_g, sem_i):
    cid = lax.axis_index("c")
    sid = lax.axis_index("s")
    w = cid * NS + sid
    base = sid * ROWS_PER_TILE

    _fill(rows_v.at[0], 0.0)
    for b in range(ROWS_PER_TILE // K):
      pltpu.sync_copy(rows_v.at[0], agg_sh.at[pl.ds(base + b * K, K)])
    plsc.subcore_barrier()

    pltpu.sync_copy(dst_hbm.at[w], dst_v)
    pltpu.sync_copy(src_hbm.at[w, 0], sidx_v.at[0])
    pltpu.async_copy(h_hbm.at[sidx_v.at[0]], rows_v.at[0], sem_g)
    pltpu.async_copy(src_hbm.at[w, 1], sidx_v.at[1], sem_i)

    @pl.loop(0, n_chunks)
    def _(j):
      cur = lax.rem(j, 2)
      nxt = lax.rem(j + 1, 2)
      pltpu.make_async_copy(h_hbm.at[sidx_v.at[cur]], rows_v.at[cur],
                            sem_g).wait()

      @pl.when(j + 1 < n_chunks)
      def _():
        pltpu.make_async_copy(src_hbm.at[w, 0], sidx_v.at[nxt],
                              sem_i).wait()
        pltpu.async_copy(h_hbm.at[sidx_v.at[nxt]], rows_v.at[nxt], sem_g)

        @pl.when(j + 2 < n_chunks)
        def _():
          pltpu.async_copy(src_hbm.at[w, j + 2], sidx_v.at[cur], sem_i)

      pltpu.sync_copy(rows_v.at[cur], agg_sh.at[dst_v.at[j]], add=True)

    plsc.subcore_barrier()
    pltpu.sync_copy(agg_sh.at[pl.ds(base, ROWS_PER_TILE)],
                    agg_out.at[cid, pl.ds(base, ROWS_PER_TILE)])

  return pl.kernel(
      body,
      out_type=jax.ShapeDtypeStruct((NC, NPAD, D), jnp.float32),
      mesh=mesh,
      scratch_types=[
          pltpu.VMEM((2, K), jnp.int32),
          pltpu.VMEM((n_chunks, K), jnp.int32),
          pltpu.VMEM((2, K, D), jnp.float32),
          pltpu.VMEM_SHARED((NPAD, D), jnp.float32),
          pltpu.SemaphoreType.DMA,
          pltpu.SemaphoreType.DMA,
      ],
  )


def _make_sc_count(n_chunks):
  mesh = plsc.VectorSubcoreMesh(core_axis_name="c", subcore_axis_name="s")

  def body(dst_hbm, cnt_out, dst_v, buf_v, cnt_sh):
    cid = lax.axis_index("c")
    sid = lax.axis_index("s")
    w = cid * NS + sid
    base = sid * ROWS_PER_TILE

    _fill(buf_v, 0.0)
    for b in range(ROWS_PER_TILE // K):
      pltpu.sync_copy(buf_v, cnt_sh.at[pl.ds(base + b * K, K)])
    plsc.subcore_barrier()

    _fill(buf_v, 1.0)
    pltpu.sync_copy(dst_hbm.at[w], dst_v)

    @pl.loop(0, n_chunks)
    def _(j):
      pltpu.sync_copy(buf_v, cnt_sh.at[dst_v.at[j]], add=True)

    plsc.subcore_barrier()
    pltpu.sync_copy(cnt_sh.at[pl.ds(base, ROWS_PER_TILE)],
                    cnt_out.at[cid, pl.ds(base, ROWS_PER_TILE)])

  return pl.kernel(
      body,
      out_type=jax.ShapeDtypeStruct((NC, NPAD, D), jnp.float32),
      mesh=mesh,
      scratch_types=[
          pltpu.VMEM((n_chunks, K), jnp.int32),
          pltpu.VMEM((K, D), jnp.float32),
          pltpu.VMEM_SHARED((NPAD, D), jnp.float32),
      ],
  )


_B = 1024


def _tc_layer_body(a0, a1, c0, c1, h, wl, bl, wr, g, be, out):
  cnt = c0[...][:, 0:1] + c1[...][:, 0:1]
  inv = 1.0 / jnp.maximum(cnt, 1.0)
  mean = (a0[...] + a1[...]) * inv
  z = (jnp.dot(mean, wl[...], preferred_element_type=jnp.float32) + bl[...]
       + jnp.dot(h[...], wr[...], preferred_element_type=jnp.float32))
  scale = g[...] * lax.rsqrt(jnp.float32(1.0 + EPS))
  z = z * scale + be[...]
  out[...] = jnp.maximum(z, 0.0) + h[...]


def _tc_final_body(a0, a1, c0, c1, h, wl, bl, wr, out):
  cnt = c0[...][:, 0:1] + c1[...][:, 0:1]
  inv = 1.0 / jnp.maximum(cnt, 1.0)
  mean = (a0[...] + a1[...]) * inv
  out[...] = (jnp.dot(mean, wl[...], preferred_element_type=jnp.float32)
              + bl[...]
              + jnp.dot(h[...], wr[...], preferred_element_type=jnp.float32))


def _row_spec():
  return pl.BlockSpec((_B, D), lambda i: (i, 0))


def _cnt_spec():
  return pl.BlockSpec((_B, CW), lambda i: (i, 0))


def _full_spec():
  return pl.BlockSpec((D, D), lambda i: (0, 0))


def _vec_spec():
  return pl.BlockSpec((1, D), lambda i: (0, 0))


def _make_tc_layer():
  return pl.pallas_call(
      _tc_layer_body,
      grid=(NPAD // _B,),
      in_specs=[_row_spec(), _row_spec(), _cnt_spec(), _cnt_spec(), _row_spec(),
                _full_spec(), _vec_spec(), _full_spec(), _vec_spec(), _vec_spec()],
      out_specs=_row_spec(),
      out_shape=jax.ShapeDtypeStruct((NPAD, D), jnp.float32),
  )


def _make_tc_final():
  return pl.pallas_call(
      _tc_final_body,
      grid=(NPAD // _B,),
      in_specs=[_row_spec(), _row_spec(), _cnt_spec(), _cnt_spec(), _row_spec(),
                _full_spec(), _vec_spec(), _full_spec()],
      out_specs=_row_spec(),
      out_shape=jax.ShapeDtypeStruct((NPAD, D), jnp.float32),
  )


def kernel(x, edge_index, Wl0, bl0, Wr0, Wl1, bl1, Wr1, Wl2, bl2, Wr2,
           g0, be0, g1, be1):
  e = edge_index.shape[1]
  n_chunks = -(-e // (NW * K))
  e_pad = NW * n_chunks * K
  src = jnp.concatenate(
      [edge_index[0], jnp.zeros((e_pad - e,), jnp.int32)]).reshape(NW, n_chunks, K)
  trash = TRASH + jnp.arange(e_pad - e, dtype=jnp.int32) % (NPAD - TRASH)
  dst = jnp.concatenate([edge_index[1], trash]).reshape(NW, n_chunks, K)
  xp = jnp.concatenate([x, jnp.zeros((NPAD - N, D), x.dtype)], axis=0)

  sc_agg = _make_sc_agg(n_chunks)
  sc_count = _make_sc_count(n_chunks)
  tc_layer = _make_tc_layer()
  tc_final = _make_tc_final()

  bl0r, g0r, be0r = bl0.reshape(1, D), g0.reshape(1, D), be0.reshape(1, D)
  bl1r, g1r, be1r = bl1.reshape(1, D), g1.reshape(1, D), be1.reshape(1, D)
  wl2p = jnp.pad(Wl2, ((0, 0), (0, D - Wl2.shape[1])))
  wr2p = jnp.pad(Wr2, ((0, 0), (0, D - Wr2.shape[1])))
  bl2p = jnp.pad(bl2, (0, D - bl2.shape[0])).reshape(1, D)

  cntw = sc_count(dst)
  cnt = cntw[:, :, :CW]
  agg = sc_agg(xp, src, dst)
  h1 = tc_layer(agg[0], agg[1], cnt[0], cnt[1], xp, Wl0, bl0r, Wr0, g0r, be0r)
  agg1 = sc_agg(h1, src, dst)
  h2 = tc_layer(agg1[0], agg1[1], cnt[0], cnt[1], h1, Wl1, bl1r, Wr1, g1r, be1r)
  agg2 = sc_agg(h2, src, dst)
  outp = tc_final(agg2[0], agg2[1], cnt[0], cnt[1], h2, wl2p, bl2p, wr2p)
  return outp[:N, :2]


# speedup vs baseline: 1.5555x; 1.0024x over previous
import jax
import jax.numpy as jnp
from jax import lax
from jax.experimental import pallas as pl
from jax.experimental.pallas import tpu as pltpu
from jax.experimental.pallas import tpu_sc as plsc

N = 10000
D = 128
EPS = 1e-5

NC = 2
NS = 16
L = 16
NW = NC * NS
K = 128
CW = 8

NPAD = 10240
ROWS_PER_TILE = NPAD // NS
TRASH = N + 64


def _fill(ref, value):
  @pl.loop(0, K)
  def _(i):
    for j in range(D // L):
      ref[i, pl.ds(j * L, L)] = jnp.full((L,), value, jnp.float32)


def _make_sc_agg(n_chunks):
  mesh = plsc.VectorSubcoreMesh(core_axis_name="c", subcore_axis_name="s")

  def body(h_hbm, src_hbm, dst_hbm, agg_out,
           sidx_v, dst_v, rows_v, agg_sh, sem_g, sem_i):
    cid = lax.axis_index("c")
    sid = lax.axis_index("s")
    w = cid * NS + sid
    base = sid * ROWS_PER_TILE

    _fill(rows_v.at[0], 0.0)
    for b in range(ROWS_PER_TILE // K):
      pltpu.async_copy(rows_v.at[0], agg_sh.at[pl.ds(base + b * K, K)], sem_g)
    pltpu.sync_copy(dst_hbm.at[w], dst_v)
    for b in range(ROWS_PER_TILE // K):
      pltpu.make_async_copy(rows_v.at[0], agg_sh.at[pl.ds(base + b * K, K)],
                            sem_g).wait()
    plsc.subcore_barrier()
    pltpu.sync_copy(src_hbm.at[w, 0], sidx_v.at[0])
    pltpu.async_copy(h_hbm.at[sidx_v.at[0]], rows_v.at[0], sem_g)
    pltpu.async_copy(src_hbm.at[w, 1], sidx_v.at[1], sem_i)

    @pl.loop(0, n_chunks)
    def _(j):
      cur = lax.rem(j, 2)
      nxt = lax.rem(j + 1, 2)
      pltpu.make_async_copy(h_hbm.at[sidx_v.at[cur]], rows_v.at[cur],
                            sem_g).wait()

      @pl.when(j + 1 < n_chunks)
      def _():
        pltpu.make_async_copy(src_hbm.at[w, 0], sidx_v.at[nxt],
                              sem_i).wait()
        pltpu.async_copy(h_hbm.at[sidx_v.at[nxt]], rows_v.at[nxt], sem_g)

        @pl.when(j + 2 < n_chunks)
        def _():
          pltpu.async_copy(src_hbm.at[w, j + 2], sidx_v.at[cur], sem_i)

      pltpu.sync_copy(rows_v.at[cur], agg_sh.at[dst_v.at[j]], add=True)

    plsc.subcore_barrier()
    pltpu.sync_copy(agg_sh.at[pl.ds(base, ROWS_PER_TILE)],
                    agg_out.at[cid, pl.ds(base, ROWS_PER_TILE)])

  return pl.kernel(
      body,
      out_type=jax.ShapeDtypeStruct((NC, NPAD, D), jnp.float32),
      mesh=mesh,
      scratch_types=[
          pltpu.VMEM((2, K), jnp.int32),
          pltpu.VMEM((n_chunks, K), jnp.int32),
          pltpu.VMEM((2, K, D), jnp.float32),
          pltpu.VMEM_SHARED((NPAD, D), jnp.float32),
          pltpu.SemaphoreType.DMA,
          pltpu.SemaphoreType.DMA,
      ],
  )


def _make_sc_count(n_chunks):
  mesh = plsc.VectorSubcoreMesh(core_axis_name="c", subcore_axis_name="s")

  def body(dst_hbm, cnt_out, dst_v, buf_v, cnt_sh):
    cid = lax.axis_index("c")
    sid = lax.axis_index("s")
    w = cid * NS + sid
    base = sid * ROWS_PER_TILE

    _fill(buf_v, 0.0)
    for b in range(ROWS_PER_TILE // K):
      pltpu.sync_copy(buf_v, cnt_sh.at[pl.ds(base + b * K, K)])
    plsc.subcore_barrier()

    _fill(buf_v, 1.0)
    pltpu.sync_copy(dst_hbm.at[w], dst_v)

    @pl.loop(0, n_chunks)
    def _(j):
      pltpu.sync_copy(buf_v, cnt_sh.at[dst_v.at[j]], add=True)

    plsc.subcore_barrier()
    pltpu.sync_copy(cnt_sh.at[pl.ds(base, ROWS_PER_TILE)],
                    cnt_out.at[cid, pl.ds(base, ROWS_PER_TILE)])

  return pl.kernel(
      body,
      out_type=jax.ShapeDtypeStruct((NC, NPAD, D), jnp.float32),
      mesh=mesh,
      scratch_types=[
          pltpu.VMEM((n_chunks, K), jnp.int32),
          pltpu.VMEM((K, D), jnp.float32),
          pltpu.VMEM_SHARED((NPAD, D), jnp.float32),
      ],
  )


_B = 1024


def _tc_layer_body(a0, a1, c0, c1, h, wl, bl, wr, g, be, out):
  cnt = c0[...][:, 0:1] + c1[...][:, 0:1]
  inv = 1.0 / jnp.maximum(cnt, 1.0)
  mean = (a0[...] + a1[...]) * inv
  z = (jnp.dot(mean, wl[...], preferred_element_type=jnp.float32) + bl[...]
       + jnp.dot(h[...], wr[...], preferred_element_type=jnp.float32))
  scale = g[...] * lax.rsqrt(jnp.float32(1.0 + EPS))
  z = z * scale + be[...]
  out[...] = jnp.maximum(z, 0.0) + h[...]


def _tc_final_body(a0, a1, c0, c1, h, wl, bl, wr, out):
  cnt = c0[...][:, 0:1] + c1[...][:, 0:1]
  inv = 1.0 / jnp.maximum(cnt, 1.0)
  mean = (a0[...] + a1[...]) * inv
  out[...] = (jnp.dot(mean, wl[...], preferred_element_type=jnp.float32)
              + bl[...]
              + jnp.dot(h[...], wr[...], preferred_element_type=jnp.float32))


def _row_spec():
  return pl.BlockSpec((_B, D), lambda i: (i, 0))


def _cnt_spec():
  return pl.BlockSpec((_B, CW), lambda i: (i, 0))


def _full_spec():
  return pl.BlockSpec((D, D), lambda i: (0, 0))


def _vec_spec():
  return pl.BlockSpec((1, D), lambda i: (0, 0))


def _make_tc_layer():
  return pl.pallas_call(
      _tc_layer_body,
      grid=(NPAD // _B,),
      in_specs=[_row_spec(), _row_spec(), _cnt_spec(), _cnt_spec(), _row_spec(),
                _full_spec(), _vec_spec(), _full_spec(), _vec_spec(), _vec_spec()],
      out_specs=_row_spec(),
      out_shape=jax.ShapeDtypeStruct((NPAD, D), jnp.float32),
  )


def _make_tc_final():
  return pl.pallas_call(
      _tc_final_body,
      grid=(NPAD // _B,),
      in_specs=[_row_spec(), _row_spec(), _cnt_spec(), _cnt_spec(), _row_spec(),
                _full_spec(), _vec_spec(), _full_spec()],
      out_specs=_row_spec(),
      out_shape=jax.ShapeDtypeStruct((NPAD, D), jnp.float32),
  )


def kernel(x, edge_index, Wl0, bl0, Wr0, Wl1, bl1, Wr1, Wl2, bl2, Wr2,
           g0, be0, g1, be1):
  e = edge_index.shape[1]
  n_chunks = -(-e // (NW * K))
  e_pad = NW * n_chunks * K
  src = jnp.concatenate(
      [edge_index[0], jnp.zeros((e_pad - e,), jnp.int32)]).reshape(NW, n_chunks, K)
  trash = TRASH + jnp.arange(e_pad - e, dtype=jnp.int32) % (NPAD - TRASH)
  dst = jnp.concatenate([edge_index[1], trash]).reshape(NW, n_chunks, K)
  xp = jnp.concatenate([x, jnp.zeros((NPAD - N, D), x.dtype)], axis=0)

  sc_agg = _make_sc_agg(n_chunks)
  sc_count = _make_sc_count(n_chunks)
  tc_layer = _make_tc_layer()
  tc_final = _make_tc_final()

  bl0r, g0r, be0r = bl0.reshape(1, D), g0.reshape(1, D), be0.reshape(1, D)
  bl1r, g1r, be1r = bl1.reshape(1, D), g1.reshape(1, D), be1.reshape(1, D)
  wl2p = jnp.pad(Wl2, ((0, 0), (0, D - Wl2.shape[1])))
  wr2p = jnp.pad(Wr2, ((0, 0), (0, D - Wr2.shape[1])))
  bl2p = jnp.pad(bl2, (0, D - bl2.shape[0])).reshape(1, D)

  cntw = sc_count(dst)
  cnt = cntw[:, :, :CW]
  agg = sc_agg(xp, src, dst)
  h1 = tc_layer(agg[0], agg[1], cnt[0], cnt[1], xp, Wl0, bl0r, Wr0, g0r, be0r)
  agg1 = sc_agg(h1, src, dst)
  h2 = tc_layer(agg1[0], agg1[1], cnt[0], cnt[1], h1, Wl1, bl1r, Wr1, g1r, be1r)
  agg2 = sc_agg(h2, src, dst)
  outp = tc_final(agg2[0], agg2[1], cnt[0], cnt[1], h2, wl2p, bl2p, wr2p)
  return outp[:N, :2]
